# baseline (speedup 1.0000x reference)
import functools

import jax
import jax.numpy as jnp
from jax import lax
from jax.experimental import pallas as pl
from jax.experimental.pallas import tpu as pltpu
from jax.experimental.pallas import tpu_sc as plsc

N_ROWS = 16384
N_COLS = 128
L = 16
NC, NS = 2, 16
NW = NC * NS
RPW = N_ROWS // NW
GROUPS = RPW // L


def _ifull(v):
    return jnp.full((L,), v, jnp.int32)


_mesh = plsc.VectorSubcoreMesh(core_axis_name="c", subcore_axis_name="s")


@functools.partial(
    pl.kernel,
    mesh=_mesh,
    out_type=jax.ShapeDtypeStruct((N_ROWS, 4), jnp.float32),
    compiler_params=pltpu.CompilerParams(
        needs_layout_passes=False, use_tc_tiling_on_sc=False
    ),
    scratch_types=[
        pltpu.VMEM((RPW, N_COLS), jnp.float32),
        pltpu.VMEM((RPW, 4), jnp.float32),
    ],
)
def _select_deepjets(x_hbm, out_hbm, rows_v, out_v):
    wid = lax.axis_index("s") * NC + lax.axis_index("c")
    base = wid * RPW
    iota = lax.iota(jnp.int32, L)

    pltpu.sync_copy(x_hbm.at[pl.ds(base, RPW)], rows_v)

    col0, col1, col2, col3 = _ifull(0), _ifull(1), _ifull(2), _ifull(3)
    col4, col5, col6, col7 = _ifull(4), _ifull(5), _ifull(6), _ifull(7)

    @pl.loop(0, GROUPS)
    def _group(g):
        ridx = iota + g * L
        b = plsc.load_gather(rows_v, [ridx, col4])
        cvb = plsc.load_gather(rows_v, [ridx, col5])
        cvl = plsc.load_gather(rows_v, [ridx, col6])
        qg = plsc.load_gather(rows_v, [ridx, col7])
        c = b / (1.0 / cvb - 1.0)
        t = c / cvl - c
        plsc.store_scatter(out_v, [ridx, col0], b)
        plsc.store_scatter(out_v, [ridx, col1], c)
        plsc.store_scatter(out_v, [ridx, col2], (1.0 - qg) * t)
        plsc.store_scatter(out_v, [ridx, col3], qg * t)

    pltpu.sync_copy(out_v, out_hbm.at[pl.ds(base, RPW)])


def kernel(x):
    return _select_deepjets(x)


# speedup vs baseline: 1.0703x; 1.0703x over previous
import functools

import jax
import jax.numpy as jnp
from jax import lax
from jax.experimental import pallas as pl
from jax.experimental.pallas import tpu as pltpu
from jax.experimental.pallas import tpu_sc as plsc
from jax.experimental.layout import Format, Layout, with_layout_constraint

N_ROWS = 16384
L = 16
NC, NS = 2, 16
NW = NC * NS
RPW = N_ROWS // NW
GROUPS = RPW // L
CHUNK = 128
NCHUNK = RPW // CHUNK


def _ifull(v):
    return jnp.full((L,), v, jnp.int32)


_mesh = plsc.VectorSubcoreMesh(core_axis_name="c", subcore_axis_name="s")


@functools.partial(
    pl.kernel,
    mesh=_mesh,
    out_type=jax.ShapeDtypeStruct((N_ROWS, 4), jnp.float32),
    compiler_params=pltpu.CompilerParams(
        needs_layout_passes=False, use_tc_tiling_on_sc=False
    ),
    scratch_types=[
        pltpu.VMEM((NCHUNK, CHUNK), jnp.int32),
        pltpu.VMEM((RPW, L), jnp.float32),
        pltpu.VMEM((RPW, 4), jnp.float32),
        pltpu.SemaphoreType.DMA,
    ],
)
def _select_deepjets(x_hbm, out_hbm, idx_v, jets_v, out_v, sem):
    wid = lax.axis_index("s") * NC + lax.axis_index("c")
    base = wid * RPW
    iota = lax.iota(jnp.int32, L)

    @pl.loop(0, NCHUNK)
    def _fill(j):
        @pl.loop(0, CHUNK // L)
        def _fill16(k):
            idx_v[j, pl.ds(k * L, L)] = (base + j * CHUNK + k * L + iota) * 8

    copies = []
    for j in range(NCHUNK):
        copies.append(
            pltpu.make_async_copy(
                x_hbm.at[idx_v.at[j]],
                jets_v.at[pl.ds(j * CHUNK, CHUNK)],
                sem,
            )
        )
    for c in copies:
        c.start()
    for c in copies:
        c.wait()

    col0, col1, col2, col3 = _ifull(0), _ifull(1), _ifull(2), _ifull(3)
    col4, col5, col6, col7 = _ifull(4), _ifull(5), _ifull(6), _ifull(7)

    @pl.loop(0, GROUPS)
    def _group(g):
        ridx = iota + g * L
        b = plsc.load_gather(jets_v, [ridx, col4])
        cvb = plsc.load_gather(jets_v, [ridx, col5])
        cvl = plsc.load_gather(jets_v, [ridx, col6])
        qg = plsc.load_gather(jets_v, [ridx, col7])
        c = b / (1.0 / cvb - 1.0)
        t = c / cvl - c
        plsc.store_scatter(out_v, [ridx, col0], b)
        plsc.store_scatter(out_v, [ridx, col1], c)
        plsc.store_scatter(out_v, [ridx, col2], (1.0 - qg) * t)
        plsc.store_scatter(out_v, [ridx, col3], qg * t)

    pltpu.sync_copy(out_v, out_hbm.at[pl.ds(base, RPW)])


def kernel(x):
    xt = x.reshape(N_ROWS * 8, L)
    xt = with_layout_constraint(
        xt, Layout(major_to_minor=(0, 1), tiling=((8,),))
    )
    return _select_deepjets(xt)


# speedup vs baseline: 1.7171x; 1.6042x over previous
import functools

import jax
import jax.numpy as jnp
from jax import lax
from jax.experimental import pallas as pl
from jax.experimental.pallas import tpu as pltpu
from jax.experimental.pallas import tpu_sc as plsc

N_ROWS = 16384
L = 16
NC, NS = 2, 16
NW = NC * NS
RPW = N_ROWS // NW
GROUPS = RPW // L
CHUNK = 128
NCHUNK = RPW // CHUNK
TILE = 128
TPW = RPW // TILE


def _ifull(v):
    return jnp.full((L,), v, jnp.int32)


_mesh = plsc.VectorSubcoreMesh(core_axis_name="c", subcore_axis_name="s")


@functools.partial(
    pl.kernel,
    mesh=_mesh,
    out_type=jax.ShapeDtypeStruct((N_ROWS // TILE * 4, TILE), jnp.float32),
    compiler_params=pltpu.CompilerParams(
        needs_layout_passes=False, use_tc_tiling_on_sc=False
    ),
    scratch_types=[
        pltpu.VMEM((NCHUNK, CHUNK), jnp.int32),
        pltpu.VMEM((RPW, L), jnp.float32),
        pltpu.VMEM((TPW * 4, TILE), jnp.float32),
        pltpu.SemaphoreType.DMA,
    ],
)
def _select_deepjets(x_hbm, out_hbm, idx_v, jets_v, out_v, sem):
    wid = lax.axis_index("s") * NC + lax.axis_index("c")
    base = wid * RPW
    iota = lax.iota(jnp.int32, L)

    @pl.loop(0, NCHUNK)
    def _fill(j):
        @pl.loop(0, CHUNK // L)
        def _fill16(k):
            idx_v[j, pl.ds(k * L, L)] = (base + j * CHUNK + k * L + iota) * 8

    copies = []
    for j in range(NCHUNK):
        copies.append(
            pltpu.make_async_copy(
                x_hbm.at[idx_v.at[j]],
                jets_v.at[pl.ds(j * CHUNK, CHUNK)],
                sem,
            )
        )
    for c in copies:
        c.start()
    for c in copies:
        c.wait()

    col4, col5, col6, col7 = _ifull(4), _ifull(5), _ifull(6), _ifull(7)

    @pl.loop(0, GROUPS)
    def _group(g):
        ridx = iota + g * L
        b = plsc.load_gather(jets_v, [ridx, col4])
        cvb = plsc.load_gather(jets_v, [ridx, col5])
        cvl = plsc.load_gather(jets_v, [ridx, col6])
        qg = plsc.load_gather(jets_v, [ridx, col7])
        c = b / (1.0 / cvb - 1.0)
        t = c / cvl - c
        tl = g // (TILE // L)
        r_off = (g % (TILE // L)) * L
        out_v[tl * 4 + 0, pl.ds(r_off, L)] = b
        out_v[tl * 4 + 1, pl.ds(r_off, L)] = c
        out_v[tl * 4 + 2, pl.ds(r_off, L)] = (1.0 - qg) * t
        out_v[tl * 4 + 3, pl.ds(r_off, L)] = qg * t

    pltpu.sync_copy(out_v, out_hbm.at[pl.ds(TPW * 4 * wid, TPW * 4)])


def kernel(x):
    xt = x.reshape(N_ROWS * 8, L)
    out = _select_deepjets(xt)
    return (
        out.reshape(N_ROWS // TILE, 4, TILE)
        .transpose(0, 2, 1)
        .reshape(N_ROWS, 4)
    )
